# baseline (speedup 1.0000x reference)
import functools

import jax
import jax.numpy as jnp
from jax import lax
from jax.experimental import pallas as pl
from jax.experimental.pallas import tpu as pltpu
from jax.experimental.pallas import tpu_sc as plsc

B = 4096
L = 200
DIM = 32
N_LABELS = 10
VOCAB = 1000000
DP = 16

NC = 2
NS = 16
NW = NC * NS
ROWS_PER_W = B // NW
CHUNK_ROWS = 16
CHUNK_TOKS = CHUNK_ROWS * L
N_CHUNKS = ROWS_PER_W // CHUNK_ROWS
GATHER_N = 128
N_G = CHUNK_TOKS // GATHER_N

PACK = 128 // DP
SLAB = VOCAB // PACK
MB = 1000


def _tw_body(*refs):
    t_refs, w_ref, o_ref = refs[:PACK], refs[PACK], refs[PACK + 1]
    parts = [jnp.dot(t[...], w_ref[...], preferred_element_type=jnp.float32)
             for t in t_refs]
    o_ref[...] = jnp.concatenate(parts, axis=1)


def _tw(tok_embed, Wp):
    in_specs = [
        pl.BlockSpec((MB, DIM), functools.partial(lambda q, j: (q * (SLAB // MB) + j, 0), q))
        for q in range(PACK)
    ]
    in_specs.append(pl.BlockSpec((DIM, DP), lambda j: (0, 0)))
    return pl.pallas_call(
        _tw_body,
        grid=(SLAB // MB,),
        in_specs=in_specs,
        out_specs=pl.BlockSpec((MB, 128), lambda j: (j, 0)),
        out_shape=jax.ShapeDtypeStruct((SLAB, 128), jnp.float32),
    )(*([tok_embed] * PACK), Wp)


def _pool_body(x_hbm, tw_hbm, bias_hbm, out_hbm, idx_v, rows_v, acc_v, bias_v,
               sem):
    wid = lax.axis_index("s") * NC + lax.axis_index("c")
    pltpu.sync_copy(bias_hbm, bias_v)

    def chunk_body(c, _):
        row_base = wid * ROWS_PER_W + c * CHUNK_ROWS
        tok_base = row_base * L
        pltpu.sync_copy(x_hbm.at[pl.ds(tok_base, CHUNK_TOKS)], idx_v)

        def remap_body(s, _):
            v = idx_v[pl.ds(s * 16, 16)]
            q = jnp.zeros((16,), jnp.int32)
            for t in range(1, PACK):
                q = q + jnp.where(v >= t * SLAB, 1, 0).astype(jnp.int32)
            idx_v[pl.ds(s * 16, 16)] = v * PACK - q * (SLAB * PACK - 1)
            return 0

        lax.fori_loop(0, CHUNK_TOKS // 16, remap_body, 0, unroll=4)

        copies = []
        for j in range(N_G):
            copies.append(pltpu.async_copy(
                tw_hbm.at[idx_v.at[pl.ds(j * GATHER_N, GATHER_N)]],
                rows_v.at[pl.ds(j * GATHER_N, GATHER_N)],
                sem))
        for cp in copies:
            cp.wait()

        def row_body(r, _):
            def tok_body(t, a):
                return a + rows_v[r * L + t, :]
            a = lax.fori_loop(0, L, tok_body, bias_v[...], unroll=8)
            acc_v[r, :] = a
            return 0

        lax.fori_loop(0, CHUNK_ROWS, row_body, 0)
        pltpu.sync_copy(acc_v, out_hbm.at[pl.ds(row_base, CHUNK_ROWS)])
        return 0

    lax.fori_loop(0, N_CHUNKS, chunk_body, 0)


_pool = functools.partial(
    pl.kernel,
    mesh=plsc.VectorSubcoreMesh(core_axis_name="c", subcore_axis_name="s"),
    compiler_params=pltpu.CompilerParams(use_tc_tiling_on_sc=False),
    out_type=jax.ShapeDtypeStruct((B, DP), jnp.float32),
    scratch_types=[
        pltpu.VMEM((CHUNK_TOKS,), jnp.int32),
        pltpu.VMEM((CHUNK_TOKS, DP), jnp.float32),
        pltpu.VMEM((CHUNK_ROWS, DP), jnp.float32),
        pltpu.VMEM((DP,), jnp.float32),
        pltpu.SemaphoreType.DMA,
    ],
)(_pool_body)


def kernel(x, seg, mask, tok_embed, W, b):
    Wp = jnp.pad(W, ((0, 0), (0, DP - N_LABELS)))
    bp = jnp.pad(b, (0, DP - N_LABELS))
    tw128 = _tw(tok_embed, Wp)
    tw = tw128.reshape(VOCAB, DP)
    x_flat = x.reshape(-1).astype(jnp.int32)
    pooled = _pool(x_flat, tw, bp)
    return pooled[:, :N_LABELS]


# speedup vs baseline: 1.7089x; 1.7089x over previous
import functools

import jax
import jax.numpy as jnp
from jax import lax
from jax.experimental import pallas as pl
from jax.experimental.pallas import tpu as pltpu
from jax.experimental.pallas import tpu_sc as plsc

B = 4096
L = 200
DIM = 32
N_LABELS = 10
VOCAB = 1000000
DP = 16

NC = 2
NS = 16
NW = NC * NS
ROWS_PER_W = B // NW
CHUNK_ROWS = 16
CHUNK_TOKS = CHUNK_ROWS * L
N_CHUNKS = ROWS_PER_W // CHUNK_ROWS
GATHER_N = 128
N_G = CHUNK_TOKS // GATHER_N

PACK = 128 // DP
VBLK = 16384
SUB = VBLK // PACK
NBLK = -(-VOCAB // VBLK)
TW_ROWS = NBLK * VBLK // PACK


def _tw_body(t_ref, w_ref, o_ref):
    parts = [
        lax.dot_general(
            t_ref[:, pl.ds(m * SUB, SUB)], w_ref[...],
            dimension_numbers=(((0,), (0,)), ((), ())),
            preferred_element_type=jnp.float32)
        for m in range(PACK)
    ]
    o_ref[...] = jnp.concatenate(parts, axis=1)


def _tw(tabT, Wp):
    return pl.pallas_call(
        _tw_body,
        grid=(NBLK,),
        in_specs=[
            pl.BlockSpec((DIM, VBLK), lambda j: (0, j)),
            pl.BlockSpec((DIM, DP), lambda j: (0, 0)),
        ],
        out_specs=pl.BlockSpec((SUB, 128), lambda j: (j, 0)),
        out_shape=jax.ShapeDtypeStruct((TW_ROWS, 128), jnp.float32),
    )(tabT, Wp)


def _pool_body(x_hbm, tw_hbm, bias_hbm, out_hbm, idx_v, rows_v, acc_v, bias_v,
               sem):
    wid = lax.axis_index("s") * NC + lax.axis_index("c")
    pltpu.sync_copy(bias_hbm, bias_v)

    def chunk_body(c, _):
        row_base = wid * ROWS_PER_W + c * CHUNK_ROWS
        tok_base = row_base * L
        pltpu.sync_copy(x_hbm.at[pl.ds(tok_base, CHUNK_TOKS)], idx_v)

        def remap_body(s, _):
            v = idx_v[pl.ds(s * 16, 16)]
            c_lo = v & (VBLK - 1)
            m = c_lo >> 11
            k = c_lo & (SUB - 1)
            idx_v[pl.ds(s * 16, 16)] = v - c_lo + (k << 3) + m
            return 0

        lax.fori_loop(0, CHUNK_TOKS // 16, remap_body, 0, unroll=4)

        copies = []
        for j in range(N_G):
            copies.append(pltpu.async_copy(
                tw_hbm.at[idx_v.at[pl.ds(j * GATHER_N, GATHER_N)]],
                rows_v.at[pl.ds(j * GATHER_N, GATHER_N)],
                sem))
        for cp in copies:
            cp.wait()

        def row_body(r, _):
            def tok_body(t, a):
                return a + rows_v[r * L + t, :]
            a = lax.fori_loop(0, L, tok_body, bias_v[...], unroll=8)
            acc_v[r, :] = a
            return 0

        lax.fori_loop(0, CHUNK_ROWS, row_body, 0)
        pltpu.sync_copy(acc_v, out_hbm.at[pl.ds(row_base, CHUNK_ROWS)])
        return 0

    lax.fori_loop(0, N_CHUNKS, chunk_body, 0)


_pool = functools.partial(
    pl.kernel,
    mesh=plsc.VectorSubcoreMesh(core_axis_name="c", subcore_axis_name="s"),
    compiler_params=pltpu.CompilerParams(use_tc_tiling_on_sc=False),
    out_type=jax.ShapeDtypeStruct((B, DP), jnp.float32),
    scratch_types=[
        pltpu.VMEM((CHUNK_TOKS,), jnp.int32),
        pltpu.VMEM((CHUNK_TOKS, DP), jnp.float32),
        pltpu.VMEM((CHUNK_ROWS, DP), jnp.float32),
        pltpu.VMEM((DP,), jnp.float32),
        pltpu.SemaphoreType.DMA,
    ],
)(_pool_body)


def kernel(x, seg, mask, tok_embed, W, b):
    tabT = tok_embed.T
    Wp = jnp.pad(W, ((0, 0), (0, DP - N_LABELS)))
    bp = jnp.pad(b, (0, DP - N_LABELS))
    tw128 = _tw(tabT, Wp)
    tw = tw128.reshape(TW_ROWS * PACK, DP)
    x_flat = x.reshape(-1).astype(jnp.int32)
    pooled = _pool(x_flat, tw, bp)
    return pooled[:, :N_LABELS]


# speedup vs baseline: 1.8278x; 1.0696x over previous
import functools

import jax
import jax.numpy as jnp
from jax import lax
from jax.experimental import pallas as pl
from jax.experimental.pallas import tpu as pltpu
from jax.experimental.pallas import tpu_sc as plsc

B = 4096
L = 200
DIM = 32
N_LABELS = 10
VOCAB = 1000000
DP = 16

NC = 2
NS = 16
NW = NC * NS
ROWS_PER_W = B // NW
CHUNK_ROWS = 16
CHUNK_TOKS = CHUNK_ROWS * L
N_CHUNKS = ROWS_PER_W // CHUNK_ROWS
GATHER_N = 128
N_G = CHUNK_TOKS // GATHER_N

PACK = 128 // DP
VBLK = 16384
SUB = VBLK // PACK
NBLK = -(-VOCAB // VBLK)
TW_ROWS = NBLK * VBLK // PACK


def _tw_body(t_ref, wt_ref, o_ref):
    p = jnp.dot(wt_ref[...], t_ref[...],
                preferred_element_type=jnp.float32)
    parts = [jnp.transpose(p[:, m * SUB:(m + 1) * SUB]) for m in range(PACK)]
    o_ref[...] = jnp.concatenate(parts, axis=1)


def _tw(tabT, WpT):
    return pl.pallas_call(
        _tw_body,
        grid=(NBLK,),
        in_specs=[
            pl.BlockSpec((DIM, VBLK), lambda j: (0, j)),
            pl.BlockSpec((DP, DIM), lambda j: (0, 0)),
        ],
        out_specs=pl.BlockSpec((SUB, 128), lambda j: (j, 0)),
        out_shape=jax.ShapeDtypeStruct((TW_ROWS, 128), jnp.float32),
    )(tabT, WpT)


def _chunk_gathers(xt_hbm, tw_hbm, wid, c, stage_v, idx_v, rows_v, sem):
    row_base = wid * ROWS_PER_W + c * CHUNK_ROWS
    pltpu.sync_copy(xt_hbm.at[:, pl.ds(row_base, CHUNK_ROWS)], stage_v)

    def remap_body(g, _):
        for u in range(GATHER_N // CHUNK_ROWS):
            v = stage_v[g * (GATHER_N // CHUNK_ROWS) + u, :]
            c_lo = v & (VBLK - 1)
            m = c_lo >> 11
            k = c_lo & (SUB - 1)
            idx_v[g, pl.ds(u * CHUNK_ROWS, CHUNK_ROWS)] = \
                v - c_lo + (k << 3) + m
        return 0

    lax.fori_loop(0, N_G, remap_body, 0)

    copies = []
    for j in range(N_G):
        copies.append(pltpu.async_copy(
            tw_hbm.at[idx_v.at[j]],
            rows_v.at[pl.ds(j * GATHER_N, GATHER_N)],
            sem))
    return copies


def _chunk_reduce(out_hbm, wid, c, rows_v, acc_v, bias_v):
    row_base = wid * ROWS_PER_W + c * CHUNK_ROWS

    def row_body(r, _):
        def tok_body(t, a):
            return a + rows_v[t * CHUNK_ROWS + r, :]
        a = lax.fori_loop(0, L, tok_body, bias_v[...], unroll=8)
        acc_v[r, :] = a
        return 0

    lax.fori_loop(0, CHUNK_ROWS, row_body, 0)
    pltpu.sync_copy(acc_v, out_hbm.at[pl.ds(row_base, CHUNK_ROWS)])


def _pool_body(xt_hbm, tw_hbm, bias_hbm, out_hbm,
               stage_v, idx0_v, idx1_v, rows0_v, rows1_v, acc_v, bias_v, sem):
    wid = lax.axis_index("s") * NC + lax.axis_index("c")
    pltpu.sync_copy(bias_hbm, bias_v)

    idx_bufs = (idx0_v, idx1_v)
    rows_bufs = (rows0_v, rows1_v)

    pending = _chunk_gathers(xt_hbm, tw_hbm, wid, 0, stage_v, idx_bufs[0],
                             rows_bufs[0], sem)
    for c in range(N_CHUNKS):
        for cp in pending:
            cp.wait()
        if c + 1 < N_CHUNKS:
            pending = _chunk_gathers(xt_hbm, tw_hbm, wid, c + 1, stage_v,
                                     idx_bufs[(c + 1) % 2],
                                     rows_bufs[(c + 1) % 2], sem)
        _chunk_reduce(out_hbm, wid, c, rows_bufs[c % 2], acc_v, bias_v)


_pool = functools.partial(
    pl.kernel,
    mesh=plsc.VectorSubcoreMesh(core_axis_name="c", subcore_axis_name="s"),
    compiler_params=pltpu.CompilerParams(use_tc_tiling_on_sc=False),
    out_type=jax.ShapeDtypeStruct((B, DP), jnp.float32),
    scratch_types=[
        pltpu.VMEM((L, CHUNK_ROWS), jnp.int32),
        pltpu.VMEM((N_G, GATHER_N), jnp.int32),
        pltpu.VMEM((N_G, GATHER_N), jnp.int32),
        pltpu.VMEM((CHUNK_TOKS, DP), jnp.float32),
        pltpu.VMEM((CHUNK_TOKS, DP), jnp.float32),
        pltpu.VMEM((CHUNK_ROWS, DP), jnp.float32),
        pltpu.VMEM((DP,), jnp.float32),
        pltpu.SemaphoreType.DMA,
    ],
)(_pool_body)


def kernel(x, seg, mask, tok_embed, W, b):
    tabT = tok_embed.T
    WpT = jnp.pad(W, ((0, 0), (0, DP - N_LABELS))).T
    bp = jnp.pad(b, (0, DP - N_LABELS))
    tw128 = _tw(tabT, WpT)
    tw = tw128.reshape(TW_ROWS * PACK, DP)
    xt = x.T.astype(jnp.int32)
    pooled = _pool(xt, tw, bp)
    return pooled[:, :N_LABELS]
